# Initial kernel scaffold; baseline (speedup 1.0000x reference)
import jax
import jax.numpy as jnp
from jax.experimental import pallas as pl


def kernel(preds_0, preds_1, preds_2, preds_coarse_0, preds_coarse_1, preds_coarse_2, slant_0, slant_1, slant_2, slant_coarse_0, slant_coarse_1, slant_coarse_2, conf_0, conf_1, conf_coarse_0, conf_coarse_1, volume_0, target, dxygt):
    raise NotImplementedError("write your pallas kernel here")


# speedup vs baseline: 21.6358x; 21.6358x over previous
import jax
import jax.numpy as jnp
from jax.experimental import pallas as pl
from jax.experimental.pallas import tpu as pltpu

_B, _H, _W = 4, 256, 512
_D = 192
_PH, _PW = 64, 128
_CHUNK = 32
_NACC = 128

_MAX_DISP = 192.0
_EPS = 1e-6


def _robust(diff):
    x = diff * 2.0
    x = x * x * (1.0 / 1.2) + 1.0
    x = jnp.exp(0.4 * jnp.log(x))
    return (x - 1.0) * 1.5


def _pixel_kernel(t_ref,
                  p0, p1, p2, p3, p4, p5,
                  c0, c1, c2, c3,
                  dxg, dyg,
                  sx0, sy0, sx1, sy1, sx2, sy2,
                  sx3, sy3, sx4, sy4, sx5, sy5,
                  out_ref):
    step = pl.program_id(0)
    t = t_ref[...]
    mask = (t < _MAX_DISP) & (t > 0.001)
    mf = mask.astype(jnp.float32)

    accs = []
    accs.append(jnp.sum(mf))

    preds = (p0, p1, p2, p3, p4, p5)
    diffs = [jnp.abs(p[...] - t) for p in preds]
    rl = 0.0
    for d in diffs:
        rl = rl + jnp.sum(_robust(d) * mf)
    accs.append(rl)

    dxgt = dxg[...]
    dygt = dyg[...]
    slants = ((sx0, sy0), (sx1, sy1), (sx2, sy2), (sx3, sy3), (sx4, sy4), (sx5, sy5))
    s_num, s_den = [], []
    for i, (sx, sy) in enumerate(slants):
        m = mf * (diffs[i] < 1.0).astype(jnp.float32)
        sxv = sx[...]
        syv = sy[...]
        tot = 0.0
        for b in range(_B):
            cross = 0.0
            for b2 in range(_B):
                cross = cross + (jnp.abs(dxgt[b] - sxv[b2])
                                 + jnp.abs(dygt[b] - syv[b2]))
            tot = tot + jnp.sum(m[b] * cross)
        s_num.append(tot)
        s_den.append(jnp.sum(m))
    accs.extend(s_num)
    accs.extend(s_den)

    confs = (c0, c1, c2, c3)
    conf_diff_idx = (1, 2, 4, 5)
    c_num, c_den = [], []
    for cr, di in zip(confs, conf_diff_idx):
        d = diffs[di]
        closer = (d < 1.0).astype(jnp.float32)
        further = (d > 1.5).astype(jnp.float32)
        sel = closer + further
        m = mf * sel
        cv = cr[...]
        loss = jnp.maximum(1.0 - cv, 0.0) * closer + jnp.maximum(cv, 0.0) * further
        c_num.append(jnp.sum(loss * mf * sel))
        c_den.append(jnp.sum(m))
    accs.extend(c_num)
    accs.extend(c_den)

    lane = jax.lax.broadcasted_iota(jnp.int32, (1, _NACC), 1)
    row = jnp.zeros((1, _NACC), jnp.float32)
    for i, v in enumerate(accs):
        row = jnp.where(lane == i, v, row)

    @pl.when(step == 0)
    def _():
        out_ref[...] = row

    @pl.when(step != 0)
    def _():
        out_ref[...] = out_ref[...] + row


def _volume_kernel(acc_ref, t_ref, vol_ref, out_ref, s_ref):
    b = pl.program_id(0)
    j = pl.program_id(1)
    nb = pl.num_programs(0)
    nj = pl.num_programs(1)
    first = (b == 0) & (j == 0)
    last = (b == nb - 1) & (j == nj - 1)

    @pl.when(first)
    def _():
        s_ref[0] = 0.0
        s_ref[1] = 0.0
        s_ref[2] = 0.0

    t = t_ref[0]
    ph = t.shape[0] // 4
    tr = jnp.max(t.reshape(ph, 4, _W), axis=1)
    wi = jax.lax.broadcasted_iota(jnp.int32, (_W, _PW), 0)
    ci = jax.lax.broadcasted_iota(jnp.int32, (_W, _PW), 1)
    pooled = None
    for k in range(4):
        sk = (wi == 4 * ci + k).astype(jnp.float32)
        pk = jax.lax.dot(tr, sk, preferred_element_type=jnp.float32)
        pooled = pk if pooled is None else jnp.maximum(pooled, pk)

    mask = (pooled < _MAX_DISP) & (pooled > 0.001)
    mf = mask.astype(jnp.float32)
    df = jnp.floor(pooled)
    frac = pooled - df
    dfi = df.astype(jnp.int32)
    d1i = jnp.minimum(dfi + 1, _D - 1)
    low = pooled - 1.5
    up = pooled + 1.5

    vol = vol_ref[0]
    diota = jax.lax.broadcasted_iota(jnp.int32, (_D, ph, _PW), 0)
    diota_f = diota.astype(jnp.float32)
    r0 = jnp.sum(jnp.where(diota == dfi[None], vol, 0.0), axis=0)
    r1 = jnp.sum(jnp.where(diota == d1i[None], vol, 0.0), axis=0)
    win = ((low[None] <= diota_f) & (diota_f <= up[None])) | (~mask[None])
    minv = jnp.min(jnp.where(win, jnp.inf, vol), axis=0)

    phi = frac * r1 + (1.0 - frac) * r0
    gt = jnp.sum(phi * mf)
    nm = jnp.sum(jnp.maximum(1.0 - minv, 0.0) * mf)
    sm = jnp.sum(mf)

    s_ref[0] = s_ref[0] + sm
    s_ref[1] = s_ref[1] + gt
    s_ref[2] = s_ref[2] + nm

    @pl.when(last)
    def _():
        scale_l = acc_ref[0, 1] / (acc_ref[0, 0] + _EPS)
        slant_l = 0.0
        for i in range(6):
            slant_l = slant_l + acc_ref[0, 2 + i] / (acc_ref[0, 8 + i] + _EPS)
        conf_l = 0.0
        for i in range(4):
            conf_l = conf_l + acc_ref[0, 14 + i] / (acc_ref[0, 18 + i] + _EPS)
        init_l = (s_ref[1] + s_ref[2]) / (s_ref[0] + _EPS)
        out_ref[0, 0] = scale_l + init_l + slant_l + conf_l


def kernel(preds_0, preds_1, preds_2, preds_coarse_0, preds_coarse_1,
           preds_coarse_2, slant_0, slant_1, slant_2, slant_coarse_0,
           slant_coarse_1, slant_coarse_2, conf_0, conf_1, conf_coarse_0,
           conf_coarse_1, volume_0, target, dxygt):
    slants = (slant_0, slant_1, slant_2,
              slant_coarse_0, slant_coarse_1, slant_coarse_2)
    pix_inputs = [target,
                  preds_0, preds_1, preds_2,
                  preds_coarse_0, preds_coarse_1, preds_coarse_2,
                  conf_0, conf_1, conf_coarse_0, conf_coarse_1,
                  dxygt[:, 0], dxygt[:, 1]]
    for s in slants:
        pix_inputs.append(s[:, 0])
        pix_inputs.append(s[:, 1])

    n_steps = _H // _CHUNK
    in_spec = pl.BlockSpec((_B, _CHUNK, _W), lambda i: (0, i, 0))
    acc = pl.pallas_call(
        _pixel_kernel,
        grid=(n_steps,),
        in_specs=[in_spec] * len(pix_inputs),
        out_specs=pl.BlockSpec((1, _NACC), lambda i: (0, 0)),
        out_shape=jax.ShapeDtypeStruct((1, _NACC), jnp.float32),
    )(*pix_inputs)

    nj = 4
    out = pl.pallas_call(
        _volume_kernel,
        grid=(_B, nj),
        in_specs=[
            pl.BlockSpec(memory_space=pltpu.SMEM),
            pl.BlockSpec((1, _H // nj, _W), lambda b, j: (b, j, 0)),
            pl.BlockSpec((1, _D, _PH // nj, _PW), lambda b, j: (b, 0, j, 0)),
        ],
        out_specs=pl.BlockSpec(memory_space=pltpu.SMEM),
        out_shape=jax.ShapeDtypeStruct((1, 1), jnp.float32),
        scratch_shapes=[pltpu.SMEM((4,), jnp.float32)],
    )(acc, target, volume_0)

    return out[0, 0]


# speedup vs baseline: 28.5560x; 1.3198x over previous
import jax
import jax.numpy as jnp
from jax.experimental import pallas as pl
from jax.experimental.pallas import tpu as pltpu

_B, _H, _W = 4, 256, 512
_D = 192
_PH, _PW = 64, 128
_CHUNK = 64
_NACC = 128

_MAX_DISP = 192.0
_EPS = 1e-6


def _robust(diff):
    x = diff * 2.0
    x = x * x * (1.0 / 1.2) + 1.0
    x = jnp.exp(0.4 * jnp.log(x))
    return (x - 1.0) * 1.5


def _pixel_kernel(t_ref,
                  p0, p1, p2, p3, p4, p5,
                  c0, c1, c2, c3,
                  dxy_ref,
                  s0, s1, s2, s3, s4, s5,
                  out_ref):
    step = pl.program_id(0)
    t = t_ref[...]
    mask = (t < _MAX_DISP) & (t > 0.001)
    mf = mask.astype(jnp.float32)

    accs = []
    accs.append(jnp.sum(mf))

    preds = (p0, p1, p2, p3, p4, p5)
    diffs = [jnp.abs(p[...] - t) for p in preds]
    rl = 0.0
    for d in diffs:
        rl = rl + jnp.sum(_robust(d) * mf)
    accs.append(rl)

    s_num, s_den = [], []
    for i, s in enumerate((s0, s1, s2, s3, s4, s5)):
        m = mf * (diffs[i] < 1.0).astype(jnp.float32)
        tot = 0.0
        for b in range(_B):
            cross = 0.0
            for b2 in range(_B):
                cross = cross + (jnp.abs(dxy_ref[b, 0] - s[b2, 0])
                                 + jnp.abs(dxy_ref[b, 1] - s[b2, 1]))
            tot = tot + jnp.sum(m[b] * cross)
        s_num.append(tot)
        s_den.append(jnp.sum(m))
    accs.extend(s_num)
    accs.extend(s_den)

    confs = (c0, c1, c2, c3)
    conf_diff_idx = (1, 2, 4, 5)
    c_num, c_den = [], []
    for cr, di in zip(confs, conf_diff_idx):
        d = diffs[di]
        closer = (d < 1.0).astype(jnp.float32)
        further = (d > 1.5).astype(jnp.float32)
        sel = closer + further
        m = mf * sel
        cv = cr[...]
        loss = jnp.maximum(1.0 - cv, 0.0) * closer + jnp.maximum(cv, 0.0) * further
        c_num.append(jnp.sum(loss * m))
        c_den.append(jnp.sum(m))
    accs.extend(c_num)
    accs.extend(c_den)

    lane = jax.lax.broadcasted_iota(jnp.int32, (1, _NACC), 1)
    row = jnp.zeros((1, _NACC), jnp.float32)
    for i, v in enumerate(accs):
        row = jnp.where(lane == i, v, row)

    @pl.when(step == 0)
    def _():
        out_ref[...] = row

    @pl.when(step != 0)
    def _():
        out_ref[...] = out_ref[...] + row


def _volume_kernel(acc_ref, t_ref, vol_ref, out_ref, s_ref):
    b = pl.program_id(0)
    nb = pl.num_programs(0)
    first = b == 0
    last = b == nb - 1

    @pl.when(first)
    def _():
        s_ref[0] = 0.0
        s_ref[1] = 0.0
        s_ref[2] = 0.0

    t = t_ref[0]
    tr = jnp.max(t.reshape(_PH, 4, _W), axis=1)
    wi = jax.lax.broadcasted_iota(jnp.int32, (_W, _PW), 0)
    ci = jax.lax.broadcasted_iota(jnp.int32, (_W, _PW), 1)
    pooled = None
    for k in range(4):
        sk = (wi == 4 * ci + k).astype(jnp.float32)
        pk = jax.lax.dot(tr, sk, preferred_element_type=jnp.float32)
        pooled = pk if pooled is None else jnp.maximum(pooled, pk)

    mask = (pooled < _MAX_DISP) & (pooled > 0.001)
    mf = mask.astype(jnp.float32)
    df = jnp.floor(pooled)
    frac = pooled - df
    dfi = df.astype(jnp.int32)
    d1i = jnp.minimum(dfi + 1, _D - 1)
    low = pooled - 1.5
    up = pooled + 1.5

    vol = vol_ref[0]
    diota = jax.lax.broadcasted_iota(jnp.int32, (_D, _PH, _PW), 0)
    diota_f = diota.astype(jnp.float32)
    r0 = jnp.sum(jnp.where(diota == dfi[None], vol, 0.0), axis=0)
    r1 = jnp.sum(jnp.where(diota == d1i[None], vol, 0.0), axis=0)
    win = ((low[None] <= diota_f) & (diota_f <= up[None])) | (~mask[None])
    minv = jnp.min(jnp.where(win, jnp.inf, vol), axis=0)

    phi = frac * r1 + (1.0 - frac) * r0
    gt = jnp.sum(phi * mf)
    nm = jnp.sum(jnp.maximum(1.0 - minv, 0.0) * mf)
    sm = jnp.sum(mf)

    s_ref[0] = s_ref[0] + sm
    s_ref[1] = s_ref[1] + gt
    s_ref[2] = s_ref[2] + nm

    @pl.when(last)
    def _():
        scale_l = acc_ref[0, 1] / (acc_ref[0, 0] + _EPS)
        slant_l = 0.0
        for i in range(6):
            slant_l = slant_l + acc_ref[0, 2 + i] / (acc_ref[0, 8 + i] + _EPS)
        conf_l = 0.0
        for i in range(4):
            conf_l = conf_l + acc_ref[0, 14 + i] / (acc_ref[0, 18 + i] + _EPS)
        init_l = (s_ref[1] + s_ref[2]) / (s_ref[0] + _EPS)
        out_ref[0, 0] = scale_l + init_l + slant_l + conf_l


def kernel(preds_0, preds_1, preds_2, preds_coarse_0, preds_coarse_1,
           preds_coarse_2, slant_0, slant_1, slant_2, slant_coarse_0,
           slant_coarse_1, slant_coarse_2, conf_0, conf_1, conf_coarse_0,
           conf_coarse_1, volume_0, target, dxygt):
    pix_inputs = [target,
                  preds_0, preds_1, preds_2,
                  preds_coarse_0, preds_coarse_1, preds_coarse_2,
                  conf_0, conf_1, conf_coarse_0, conf_coarse_1,
                  dxygt,
                  slant_0, slant_1, slant_2,
                  slant_coarse_0, slant_coarse_1, slant_coarse_2]

    n_steps = _H // _CHUNK
    in_spec3 = pl.BlockSpec((_B, _CHUNK, _W), lambda i: (0, i, 0))
    in_spec4 = pl.BlockSpec((_B, 2, _CHUNK, _W), lambda i: (0, 0, i, 0))
    specs = [in_spec3] * 11 + [in_spec4] * 7
    acc = pl.pallas_call(
        _pixel_kernel,
        grid=(n_steps,),
        in_specs=specs,
        out_specs=pl.BlockSpec((1, _NACC), lambda i: (0, 0)),
        out_shape=jax.ShapeDtypeStruct((1, _NACC), jnp.float32),
    )(*pix_inputs)

    out = pl.pallas_call(
        _volume_kernel,
        grid=(_B,),
        in_specs=[
            pl.BlockSpec(memory_space=pltpu.SMEM),
            pl.BlockSpec((1, _H, _W), lambda b: (b, 0, 0)),
            pl.BlockSpec((1, _D, _PH, _PW), lambda b: (b, 0, 0, 0)),
        ],
        out_specs=pl.BlockSpec(memory_space=pltpu.SMEM),
        out_shape=jax.ShapeDtypeStruct((1, 1), jnp.float32),
        scratch_shapes=[pltpu.SMEM((4,), jnp.float32)],
    )(acc, target, volume_0)

    return out[0, 0]
